# baseline (speedup 1.0000x reference)
import functools

import jax
import jax.numpy as jnp
from jax import lax
from jax.experimental import pallas as pl
from jax.experimental.pallas import tpu as pltpu
from jax.experimental.pallas import tpu_sc as plsc

N = 10000
E = 320000
D = 128
H = 128
C = 64

NC = 2
NS = 16
NW = NC * NS

NP = 10240
RPS = NP // NS
B = 128
CHP = 80
SLAB = 40
EPAD = NW * CHP * B

_mesh = plsc.VectorSubcoreMesh(
    core_axis_name="c", subcore_axis_name="s", num_cores=NC, num_subcores=NS
)


@functools.partial(
    pl.kernel,
    out_type=jax.ShapeDtypeStruct((NC, 1, NP), jnp.float32),
    mesh=_mesh,
    scratch_types=[
        pltpu.VMEM((B,), jnp.float32),
        pltpu.VMEM((CHP, B), jnp.int32),
        pltpu.VMEM_SHARED((NP,), jnp.float32),
    ],
)
def _deg_kernel(dst_hbm, zero_hbm, out_hbm, ones_v, idx_v, deg_sh):
    c = lax.axis_index("c")
    s = lax.axis_index("s")
    wid = c * NS + s
    for i in range(B // 16):
        ones_v[pl.ds(i * 16, 16)] = jnp.ones((16,), jnp.float32)
    pltpu.sync_copy(zero_hbm.at[pl.ds(s * RPS, RPS)], deg_sh.at[pl.ds(s * RPS, RPS)])
    plsc.subcore_barrier()
    pltpu.sync_copy(dst_hbm.at[wid], idx_v)

    def chunk(j, carry):
        pltpu.sync_copy(ones_v, deg_sh.at[idx_v.at[j]], add=True)
        return carry

    lax.fori_loop(0, CHP, chunk, 0)
    plsc.subcore_barrier()
    pltpu.sync_copy(
        deg_sh.at[pl.ds(s * RPS, RPS)], out_hbm.at[c, 0, pl.ds(s * RPS, RPS)]
    )


def _make_agg_kernel(width):
    @functools.partial(
        pl.kernel,
        out_type=jax.ShapeDtypeStruct((NC, NP, width), jnp.float32),
        mesh=_mesh,
        scratch_types=[
            pltpu.VMEM((SLAB, B), jnp.int32),
            pltpu.VMEM((SLAB, B), jnp.int32),
            pltpu.VMEM((B, width), jnp.float32),
            pltpu.VMEM((B, width), jnp.float32),
            pltpu.VMEM_SHARED((NP, width), jnp.float32),
            pltpu.SemaphoreType.DMA,
            pltpu.SemaphoreType.DMA,
        ],
    )
    def _agg(
        y_hbm, src_hbm, dst_hbm, out_hbm, idxs_v, idxd_v, rows0_v, rows1_v,
        agg_sh, sem0, sem1,
    ):
        c = lax.axis_index("c")
        s = lax.axis_index("s")
        wid = c * NS + s
        pltpu.sync_copy(y_hbm.at[pl.ds(s * RPS, RPS)], agg_sh.at[pl.ds(s * RPS, RPS)])
        plsc.subcore_barrier()
        for half in range(CHP // SLAB):
            pltpu.sync_copy(src_hbm.at[wid, pl.ds(half * SLAB, SLAB)], idxs_v)
            pltpu.sync_copy(dst_hbm.at[wid, pl.ds(half * SLAB, SLAB)], idxd_v)
            pltpu.async_copy(y_hbm.at[idxs_v.at[0]], rows0_v, sem0)

            def pair(i, carry):
                j = 2 * i
                pltpu.async_copy(y_hbm.at[idxs_v.at[j + 1]], rows1_v, sem1)
                pltpu.make_async_copy(y_hbm.at[idxs_v.at[j]], rows0_v, sem0).wait()
                pltpu.sync_copy(rows0_v, agg_sh.at[idxd_v.at[j]], add=True)

                @pl.when(j + 2 < SLAB)
                def _():
                    pltpu.async_copy(y_hbm.at[idxs_v.at[j + 2]], rows0_v, sem0)

                pltpu.make_async_copy(y_hbm.at[idxs_v.at[j + 1]], rows1_v, sem1).wait()
                pltpu.sync_copy(rows1_v, agg_sh.at[idxd_v.at[j + 1]], add=True)
                return carry

            lax.fori_loop(0, SLAB // 2, pair, 0)
        plsc.subcore_barrier()
        pltpu.sync_copy(
            agg_sh.at[pl.ds(s * RPS, RPS)], out_hbm.at[c, pl.ds(s * RPS, RPS)]
        )

    return _agg


_agg128 = _make_agg_kernel(D)


_BR = NP // 8


def _dinv_from(degp_ref):
    i = pl.program_id(0)
    sl = pl.ds(i * _BR, _BR)
    deg = degp_ref[0, sl] + degp_ref[1, sl] + 1.0
    return lax.rsqrt(deg)


def _tc1_body(x_ref, w_ref, degp_ref, y_ref):
    dinv = _dinv_from(degp_ref)
    xw = jnp.dot(x_ref[...], w_ref[...], preferred_element_type=jnp.float32)
    y_ref[...] = xw * dinv[:, None]


def _tc2_body(aggp_ref, y1_ref, degp_ref, w2_ref, b1_ref, y2_ref):
    dinv = _dinv_from(degp_ref)
    aggt = aggp_ref[0] + aggp_ref[1] - y1_ref[...]
    h = jnp.maximum(aggt * dinv[:, None] + b1_ref[0][None, :], 0.0)
    hw = jnp.dot(h, w2_ref[...], preferred_element_type=jnp.float32)
    y2_ref[...] = jnp.concatenate(
        [hw * dinv[:, None], jnp.zeros((hw.shape[0], H - C), jnp.float32)], axis=1
    )


def _tc3_body(aggp_ref, y2_ref, degp_ref, b2_ref, z_ref):
    dinv = _dinv_from(degp_ref)
    aggt = aggp_ref[0, :, :C] + aggp_ref[1, :, :C] - y2_ref[:, :C]
    o = aggt * dinv[:, None] + b2_ref[0][None, :]
    m = jnp.max(o, axis=1, keepdims=True)
    ex = jnp.exp(o - m)
    lse = jnp.log(jnp.sum(ex, axis=1, keepdims=True))
    z_ref[...] = (o - m) - lse


def _row_spec(width):
    return pl.BlockSpec((_BR, width), lambda i: (i, 0))


_degp_spec = pl.BlockSpec((NC, NP), lambda i: (0, 0))


def _tc1(x, w1, degp):
    return pl.pallas_call(
        _tc1_body,
        grid=(NP // _BR,),
        in_specs=[
            _row_spec(D),
            pl.BlockSpec((D, H), lambda i: (0, 0)),
            _degp_spec,
        ],
        out_specs=_row_spec(H),
        out_shape=jax.ShapeDtypeStruct((NP, H), jnp.float32),
    )(x, w1, degp)


def _tc2(aggp, y1, degp, w2, b1):
    return pl.pallas_call(
        _tc2_body,
        grid=(NP // _BR,),
        in_specs=[
            pl.BlockSpec((NC, _BR, H), lambda i: (0, i, 0)),
            _row_spec(H),
            _degp_spec,
            pl.BlockSpec((H, C), lambda i: (0, 0)),
            pl.BlockSpec((1, H), lambda i: (0, 0)),
        ],
        out_specs=_row_spec(H),
        out_shape=jax.ShapeDtypeStruct((NP, H), jnp.float32),
    )(aggp, y1, degp, w2, b1)


def _tc3(aggp, y2, degp, b2):
    return pl.pallas_call(
        _tc3_body,
        grid=(NP // _BR,),
        in_specs=[
            pl.BlockSpec((NC, _BR, H), lambda i: (0, i, 0)),
            _row_spec(H),
            _degp_spec,
            pl.BlockSpec((1, C), lambda i: (0, 0)),
        ],
        out_specs=_row_spec(C),
        out_shape=jax.ShapeDtypeStruct((NP, C), jnp.float32),
    )(aggp, y2, degp, b2)


@jax.jit
def kernel(x, edge_index, W1, b1, W2, b2):
    pad = jnp.full((EPAD - E,), NP - 1, jnp.int32)
    src2d = jnp.concatenate([edge_index[0].astype(jnp.int32), pad]).reshape(
        NW, CHP, B
    )
    dst2d = jnp.concatenate([edge_index[1].astype(jnp.int32), pad]).reshape(
        NW, CHP, B
    )
    xp = jnp.pad(x, ((0, NP - N), (0, 0)))
    zeros = jnp.zeros((NP,), jnp.float32)

    degp = _deg_kernel(dst2d, zeros).reshape(NC, NP)
    y1 = _tc1(xp, W1, degp)
    aggp1 = _agg128(y1, src2d, dst2d)
    y2 = _tc2(aggp1, y1, degp, W2, b1.reshape(1, H))
    aggp2 = _agg128(y2, src2d, dst2d)
    z = _tc3(aggp2, y2, degp, b2.reshape(1, C))
    return z[:N]


# speedup vs baseline: 3.7344x; 3.7344x over previous
import functools

import jax
import jax.numpy as jnp
from jax import lax
from jax.experimental import pallas as pl
from jax.experimental.pallas import tpu as pltpu
from jax.experimental.pallas import tpu_sc as plsc

N = 10000
E = 320000
D = 128
H = 128
C = 64

NC = 2
NS = 16
NW = NC * NS

NP = 10240
RPS = NP // NS
B = 128
CHP = 80
SLAB = 40
EPAD = NW * CHP * B

_mesh = plsc.VectorSubcoreMesh(
    core_axis_name="c", subcore_axis_name="s", num_cores=NC, num_subcores=NS
)


@functools.partial(
    pl.kernel,
    out_type=jax.ShapeDtypeStruct((NC, 1, NP), jnp.float32),
    mesh=_mesh,
    scratch_types=[
        pltpu.VMEM((B,), jnp.float32),
        pltpu.VMEM((CHP, B), jnp.int32),
        pltpu.VMEM_SHARED((NP,), jnp.float32),
    ],
)
def _deg_kernel(dst_hbm, zero_hbm, out_hbm, ones_v, idx_v, deg_sh):
    c = lax.axis_index("c")
    s = lax.axis_index("s")
    wid = c * NS + s
    for i in range(B // 16):
        ones_v[pl.ds(i * 16, 16)] = jnp.ones((16,), jnp.float32)
    pltpu.sync_copy(zero_hbm.at[pl.ds(s * RPS, RPS)], deg_sh.at[pl.ds(s * RPS, RPS)])
    plsc.subcore_barrier()
    pltpu.sync_copy(dst_hbm.at[wid], idx_v)

    def chunk(j, carry):
        pltpu.sync_copy(ones_v, deg_sh.at[idx_v.at[j]], add=True)
        return carry

    lax.fori_loop(0, CHP, chunk, 0)
    plsc.subcore_barrier()
    pltpu.sync_copy(
        deg_sh.at[pl.ds(s * RPS, RPS)], out_hbm.at[c, 0, pl.ds(s * RPS, RPS)]
    )


def _make_agg_kernel(width):
    @functools.partial(
        pl.kernel,
        out_type=jax.ShapeDtypeStruct((NC, NP, width), jnp.float32),
        mesh=_mesh,
        scratch_types=[
            pltpu.VMEM((SLAB, B), jnp.int32),
            pltpu.VMEM((SLAB, B), jnp.int32),
            pltpu.VMEM((B, width), jnp.float32),
            pltpu.VMEM((B, width), jnp.float32),
            pltpu.VMEM_SHARED((NP, width), jnp.float32),
            pltpu.SemaphoreType.DMA,
            pltpu.SemaphoreType.DMA,
        ],
    )
    def _agg(
        y_hbm, src_hbm, dst_hbm, out_hbm, idxs_v, idxd_v, rows0_v, rows1_v,
        agg_sh, sem0, sem1,
    ):
        c = lax.axis_index("c")
        s = lax.axis_index("s")
        wid = c * NS + s
        pltpu.sync_copy(y_hbm.at[pl.ds(s * RPS, RPS)], agg_sh.at[pl.ds(s * RPS, RPS)])
        plsc.subcore_barrier()
        for half in range(CHP // SLAB):
            pltpu.sync_copy(src_hbm.at[wid, pl.ds(half * SLAB, SLAB)], idxs_v)
            pltpu.sync_copy(dst_hbm.at[wid, pl.ds(half * SLAB, SLAB)], idxd_v)
            pltpu.async_copy(y_hbm.at[idxs_v.at[0]], rows0_v, sem0)

            def pair(i, carry):
                j = 2 * i
                pltpu.async_copy(y_hbm.at[idxs_v.at[j + 1]], rows1_v, sem1)
                pltpu.make_async_copy(y_hbm.at[idxs_v.at[j]], rows0_v, sem0).wait()
                pltpu.sync_copy(rows0_v, agg_sh.at[idxd_v.at[j]], add=True)

                @pl.when(j + 2 < SLAB)
                def _():
                    pltpu.async_copy(y_hbm.at[idxs_v.at[j + 2]], rows0_v, sem0)

                pltpu.make_async_copy(y_hbm.at[idxs_v.at[j + 1]], rows1_v, sem1).wait()
                pltpu.sync_copy(rows1_v, agg_sh.at[idxd_v.at[j + 1]], add=True)
                return carry

            lax.fori_loop(0, SLAB // 2, pair, 0)
        plsc.subcore_barrier()
        pltpu.sync_copy(
            agg_sh.at[pl.ds(s * RPS, RPS)], out_hbm.at[c, pl.ds(s * RPS, RPS)]
        )

    return _agg


_agg128 = _make_agg_kernel(D)


_BR = NP // 8


def _dinv_from(degp_ref):
    i = pl.program_id(0)
    sl = pl.ds(i * _BR, _BR)
    deg = degp_ref[0, sl] + degp_ref[1, sl] + 1.0
    return lax.rsqrt(deg)


def _tc1_body(x_ref, w_ref, degp_ref, y_ref):
    dinv = _dinv_from(degp_ref)
    xw = jnp.dot(x_ref[...], w_ref[...], preferred_element_type=jnp.float32)
    y_ref[...] = xw * dinv[:, None]


def _tc2_body(aggp_ref, y1_ref, degp_ref, w2_ref, b1_ref, y2_ref):
    dinv = _dinv_from(degp_ref)
    aggt = aggp_ref[0] + aggp_ref[1] - y1_ref[...]
    h = jnp.maximum(aggt * dinv[:, None] + b1_ref[0][None, :], 0.0)
    hw = jnp.dot(h, w2_ref[...], preferred_element_type=jnp.float32)
    y2_ref[...] = jnp.concatenate(
        [hw * dinv[:, None], jnp.zeros((hw.shape[0], H - C), jnp.float32)], axis=1
    )


def _tc3_body(aggp_ref, y2_ref, degp_ref, b2_ref, z_ref):
    dinv = _dinv_from(degp_ref)
    aggt = aggp_ref[0, :, :C] + aggp_ref[1, :, :C] - y2_ref[:, :C]
    o = aggt * dinv[:, None] + b2_ref[0][None, :]
    m = jnp.max(o, axis=1, keepdims=True)
    ex = jnp.exp(o - m)
    lse = jnp.log(jnp.sum(ex, axis=1, keepdims=True))
    z_ref[...] = (o - m) - lse


def _row_spec(width):
    return pl.BlockSpec((_BR, width), lambda i: (i, 0))


_degp_spec = pl.BlockSpec((NC, NP), lambda i: (0, 0))


def _tc1(x, w1, degp):
    return pl.pallas_call(
        _tc1_body,
        grid=(NP // _BR,),
        in_specs=[
            _row_spec(D),
            pl.BlockSpec((D, H), lambda i: (0, 0)),
            _degp_spec,
        ],
        out_specs=_row_spec(H),
        out_shape=jax.ShapeDtypeStruct((NP, H), jnp.float32),
    )(x, w1, degp)


def _tc2(aggp, y1, degp, w2, b1):
    return pl.pallas_call(
        _tc2_body,
        grid=(NP // _BR,),
        in_specs=[
            pl.BlockSpec((NC, _BR, H), lambda i: (0, i, 0)),
            _row_spec(H),
            _degp_spec,
            pl.BlockSpec((H, C), lambda i: (0, 0)),
            pl.BlockSpec((1, H), lambda i: (0, 0)),
        ],
        out_specs=_row_spec(H),
        out_shape=jax.ShapeDtypeStruct((NP, H), jnp.float32),
    )(aggp, y1, degp, w2, b1)


def _tc3(aggp, y2, degp, b2):
    return pl.pallas_call(
        _tc3_body,
        grid=(NP // _BR,),
        in_specs=[
            pl.BlockSpec((NC, _BR, H), lambda i: (0, i, 0)),
            _row_spec(H),
            _degp_spec,
            pl.BlockSpec((1, C), lambda i: (0, 0)),
        ],
        out_specs=_row_spec(C),
        out_shape=jax.ShapeDtypeStruct((NP, C), jnp.float32),
    )(aggp, y2, degp, b2)


@jax.jit
def kernel(x, edge_index, W1, b1, W2, b2):
    ppt = (EPAD - E) // NW
    padv = (N + jnp.arange(ppt, dtype=jnp.int32) % (NP - N))[None, :]
    pad = jnp.broadcast_to(padv, (NW, ppt))
    src2d = jnp.concatenate(
        [edge_index[0].astype(jnp.int32).reshape(NW, E // NW), pad], axis=1
    ).reshape(NW, CHP, B)
    dst2d = jnp.concatenate(
        [edge_index[1].astype(jnp.int32).reshape(NW, E // NW), pad], axis=1
    ).reshape(NW, CHP, B)
    xp = jnp.pad(x, ((0, NP - N), (0, 0)))
    zeros = jnp.zeros((NP,), jnp.float32)

    degp = _deg_kernel(dst2d, zeros).reshape(NC, NP)
    y1 = _tc1(xp, W1, degp)
    aggp1 = _agg128(y1, src2d, dst2d)
    y2 = _tc2(aggp1, y1, degp, W2, b1.reshape(1, H))
    aggp2 = _agg128(y2, src2d, dst2d)
    z = _tc3(aggp2, y2, degp, b2.reshape(1, C))
    return z[:N]


# speedup vs baseline: 4.0659x; 1.0888x over previous
import functools

import jax
import jax.numpy as jnp
from jax import lax
from jax.experimental import pallas as pl
from jax.experimental.pallas import tpu as pltpu
from jax.experimental.pallas import tpu_sc as plsc

N = 10000
E = 320000
D = 128
H = 128
C = 64

NC = 2
NS = 16
NW = NC * NS

NP = 10240
RPS = NP // NS
B = 128
CHP = 80
SLAB = 40
EPAD = NW * CHP * B

_mesh = plsc.VectorSubcoreMesh(
    core_axis_name="c", subcore_axis_name="s", num_cores=NC, num_subcores=NS
)


@functools.partial(
    pl.kernel,
    out_type=jax.ShapeDtypeStruct((NC, 1, NP), jnp.float32),
    mesh=_mesh,
    scratch_types=[
        pltpu.VMEM((B,), jnp.float32),
        pltpu.VMEM((CHP, B), jnp.int32),
        pltpu.VMEM_SHARED((NP,), jnp.float32),
    ],
)
def _deg_kernel(dst_hbm, zero_hbm, out_hbm, ones_v, idx_v, deg_sh):
    c = lax.axis_index("c")
    s = lax.axis_index("s")
    wid = c * NS + s
    for i in range(B // 16):
        ones_v[pl.ds(i * 16, 16)] = jnp.ones((16,), jnp.float32)
    pltpu.sync_copy(zero_hbm.at[pl.ds(s * RPS, RPS)], deg_sh.at[pl.ds(s * RPS, RPS)])
    plsc.subcore_barrier()
    pltpu.sync_copy(dst_hbm.at[wid], idx_v)

    def chunk(j, carry):
        pltpu.sync_copy(ones_v, deg_sh.at[idx_v.at[j]], add=True)
        return carry

    lax.fori_loop(0, CHP, chunk, 0)
    plsc.subcore_barrier()
    pltpu.sync_copy(
        deg_sh.at[pl.ds(s * RPS, RPS)], out_hbm.at[c, 0, pl.ds(s * RPS, RPS)]
    )


def _make_agg_kernel(width, tc_tiling=True):
    @functools.partial(
        pl.kernel,
        out_type=jax.ShapeDtypeStruct((NC, NP, width), jnp.float32),
        mesh=_mesh,
        compiler_params=pltpu.CompilerParams(use_tc_tiling_on_sc=tc_tiling),
        scratch_types=[
            pltpu.VMEM((SLAB, B), jnp.int32),
            pltpu.VMEM((SLAB, B), jnp.int32),
            pltpu.VMEM((B, width), jnp.float32),
            pltpu.VMEM((B, width), jnp.float32),
            pltpu.VMEM_SHARED((NP, width), jnp.float32),
            pltpu.SemaphoreType.DMA,
            pltpu.SemaphoreType.DMA,
        ],
    )
    def _agg(
        y_hbm, src_hbm, dst_hbm, out_hbm, idxs_v, idxd_v, rows0_v, rows1_v,
        agg_sh, sem0, sem1,
    ):
        c = lax.axis_index("c")
        s = lax.axis_index("s")
        wid = c * NS + s
        pltpu.sync_copy(y_hbm.at[pl.ds(s * RPS, RPS)], agg_sh.at[pl.ds(s * RPS, RPS)])
        plsc.subcore_barrier()
        for half in range(CHP // SLAB):
            pltpu.sync_copy(src_hbm.at[wid, pl.ds(half * SLAB, SLAB)], idxs_v)
            pltpu.sync_copy(dst_hbm.at[wid, pl.ds(half * SLAB, SLAB)], idxd_v)
            pltpu.async_copy(y_hbm.at[idxs_v.at[0]], rows0_v, sem0)

            def pair(i, carry):
                j = 2 * i
                pltpu.async_copy(y_hbm.at[idxs_v.at[j + 1]], rows1_v, sem1)
                pltpu.make_async_copy(y_hbm.at[idxs_v.at[j]], rows0_v, sem0).wait()
                pltpu.sync_copy(rows0_v, agg_sh.at[idxd_v.at[j]], add=True)

                @pl.when(j + 2 < SLAB)
                def _():
                    pltpu.async_copy(y_hbm.at[idxs_v.at[j + 2]], rows0_v, sem0)

                pltpu.make_async_copy(y_hbm.at[idxs_v.at[j + 1]], rows1_v, sem1).wait()
                pltpu.sync_copy(rows1_v, agg_sh.at[idxd_v.at[j + 1]], add=True)
                return carry

            lax.fori_loop(0, SLAB // 2, pair, 0)
        plsc.subcore_barrier()
        pltpu.sync_copy(
            agg_sh.at[pl.ds(s * RPS, RPS)], out_hbm.at[c, pl.ds(s * RPS, RPS)]
        )

    return _agg


_agg128 = _make_agg_kernel(D)
_agg64 = _make_agg_kernel(C, tc_tiling=False)


_BR = NP // 8


def _dinv_from(degp_ref):
    i = pl.program_id(0)
    sl = pl.ds(i * _BR, _BR)
    deg = degp_ref[0, sl] + degp_ref[1, sl] + 1.0
    return lax.rsqrt(deg)


def _tc1_body(x_ref, w_ref, degp_ref, y_ref):
    dinv = _dinv_from(degp_ref)
    xw = jnp.dot(x_ref[...], w_ref[...], preferred_element_type=jnp.float32)
    y_ref[...] = xw * dinv[:, None]


def _tc2_body(aggp_ref, y1_ref, degp_ref, w2_ref, b1_ref, y2_ref):
    dinv = _dinv_from(degp_ref)
    aggt = aggp_ref[0] + aggp_ref[1] - y1_ref[...]
    h = jnp.maximum(aggt * dinv[:, None] + b1_ref[0][None, :], 0.0)
    hw = jnp.dot(h, w2_ref[...], preferred_element_type=jnp.float32)
    y2_ref[...] = hw * dinv[:, None]


def _tc3_body(aggp_ref, y2_ref, degp_ref, b2_ref, z_ref):
    dinv = _dinv_from(degp_ref)
    aggt = aggp_ref[0] + aggp_ref[1] - y2_ref[...]
    o = aggt * dinv[:, None] + b2_ref[0][None, :]
    m = jnp.max(o, axis=1, keepdims=True)
    ex = jnp.exp(o - m)
    lse = jnp.log(jnp.sum(ex, axis=1, keepdims=True))
    z_ref[...] = (o - m) - lse


def _row_spec(width):
    return pl.BlockSpec((_BR, width), lambda i: (i, 0))


_degp_spec = pl.BlockSpec((NC, NP), lambda i: (0, 0))


def _tc1(x, w1, degp):
    return pl.pallas_call(
        _tc1_body,
        grid=(NP // _BR,),
        in_specs=[
            _row_spec(D),
            pl.BlockSpec((D, H), lambda i: (0, 0)),
            _degp_spec,
        ],
        out_specs=_row_spec(H),
        out_shape=jax.ShapeDtypeStruct((NP, H), jnp.float32),
    )(x, w1, degp)


def _tc2(aggp, y1, degp, w2, b1):
    return pl.pallas_call(
        _tc2_body,
        grid=(NP // _BR,),
        in_specs=[
            pl.BlockSpec((NC, _BR, H), lambda i: (0, i, 0)),
            _row_spec(H),
            _degp_spec,
            pl.BlockSpec((H, C), lambda i: (0, 0)),
            pl.BlockSpec((1, H), lambda i: (0, 0)),
        ],
        out_specs=_row_spec(C),
        out_shape=jax.ShapeDtypeStruct((NP, C), jnp.float32),
    )(aggp, y1, degp, w2, b1)


def _tc3(aggp, y2, degp, b2):
    return pl.pallas_call(
        _tc3_body,
        grid=(NP // _BR,),
        in_specs=[
            pl.BlockSpec((NC, _BR, C), lambda i: (0, i, 0)),
            _row_spec(C),
            _degp_spec,
            pl.BlockSpec((1, C), lambda i: (0, 0)),
        ],
        out_specs=_row_spec(C),
        out_shape=jax.ShapeDtypeStruct((NP, C), jnp.float32),
    )(aggp, y2, degp, b2)


@jax.jit
def kernel(x, edge_index, W1, b1, W2, b2):
    ppt = (EPAD - E) // NW
    padv = (N + jnp.arange(ppt, dtype=jnp.int32) % (NP - N))[None, :]
    pad = jnp.broadcast_to(padv, (NW, ppt))
    src2d = jnp.concatenate(
        [edge_index[0].astype(jnp.int32).reshape(NW, E // NW), pad], axis=1
    ).reshape(NW, CHP, B)
    dst2d = jnp.concatenate(
        [edge_index[1].astype(jnp.int32).reshape(NW, E // NW), pad], axis=1
    ).reshape(NW, CHP, B)
    xp = jnp.pad(x, ((0, NP - N), (0, 0)))
    zeros = jnp.zeros((NP,), jnp.float32)

    degp = _deg_kernel(dst2d, zeros).reshape(NC, NP)
    y1 = _tc1(xp, W1, degp)
    aggp1 = _agg128(y1, src2d, dst2d)
    y2 = _tc2(aggp1, y1, degp, W2, b1.reshape(1, H))
    aggp2 = _agg64(y2, src2d, dst2d)
    z = _tc3(aggp2, y2, degp, b2.reshape(1, C))
    return z[:N]
